# Initial kernel scaffold; baseline (speedup 1.0000x reference)
import jax
import jax.numpy as jnp
from jax.experimental import pallas as pl


def kernel(eps_2d, esrc, edst, ew, ndeg, W1_0, b1_0, W2_0, b2_0, ln_s_0, ln_b_0, W1_1, b1_1, W2_1, b2_1, ln_s_1, ln_b_1):
    raise NotImplementedError("write your pallas kernel here")


# speedup vs baseline: 57.9161x; 57.9161x over previous
import functools

import jax
import jax.numpy as jnp
from jax import lax
from jax.experimental import pallas as pl
from jax.experimental.pallas import tpu as pltpu
from jax.experimental.pallas import tpu_sc as plsc

N_GRID = 224
N_NODES = N_GRID * N_GRID
E_TOTAL = N_NODES * 16
HIDDEN = 96
LANES = 16
NC, NS = 2, 16
NW = NC * NS
E_PER_W = E_TOTAL // NW
GV = E_PER_W // LANES
ROWS = E_TOTAL // 128
ROWS_PER_W = ROWS // NW
N_PER_TILE = N_NODES // NS

_mesh = plsc.VectorSubcoreMesh(core_axis_name="c", subcore_axis_name="s")
_sc_params = pltpu.CompilerParams(needs_layout_passes=False)


@functools.partial(
    pl.kernel,
    out_type=[jax.ShapeDtypeStruct((E_TOTAL,), jnp.float32),
              jax.ShapeDtypeStruct((E_TOTAL,), jnp.float32)],
    mesh=_mesh,
    scratch_types=[pltpu.VMEM((N_NODES,), jnp.float32),
                   pltpu.VMEM((E_PER_W,), jnp.int32),
                   pltpu.VMEM((E_PER_W,), jnp.float32)],
    compiler_params=_sc_params,
)
def _sc_gather(h_hbm, esrc_hbm, edst_hbm, hs_out, hd_out, h_v, idx_v, out_v):
    wid = lax.axis_index("s") * NC + lax.axis_index("c")
    base = wid * E_PER_W
    pltpu.sync_copy(h_hbm, h_v)

    def one_pass(idx_hbm, o_hbm):
        pltpu.sync_copy(idx_hbm.at[pl.ds(base, E_PER_W)], idx_v)

        def body(i, carry):
            iv = idx_v[pl.ds(i * LANES, LANES)]
            out_v[pl.ds(i * LANES, LANES)] = plsc.load_gather(h_v, [iv])
            return carry

        lax.fori_loop(0, GV, body, 0, unroll=8)
        pltpu.sync_copy(out_v, o_hbm.at[pl.ds(base, E_PER_W)])

    one_pass(esrc_hbm, hs_out)
    one_pass(edst_hbm, hd_out)


@functools.partial(
    pl.kernel,
    out_type=jax.ShapeDtypeStruct((NC, N_NODES), jnp.float32),
    mesh=_mesh,
    scratch_types=[pltpu.VMEM((ROWS_PER_W, 128), jnp.int32),
                   pltpu.VMEM((ROWS_PER_W, 128), jnp.float32),
                   pltpu.VMEM((N_PER_TILE,), jnp.float32),
                   pltpu.VMEM_SHARED((N_NODES,), jnp.float32)],
    compiler_params=_sc_params,
)
def _sc_scatter(m_hbm, edst_hbm, out_hbm, idx_v, val_v, zero_v, acc_sh):
    cid = lax.axis_index("c")
    sid = lax.axis_index("s")
    wid = sid * NC + cid

    def zbody(i, carry):
        zero_v[pl.ds(i * LANES, LANES)] = jnp.zeros((LANES,), jnp.float32)
        return carry

    lax.fori_loop(0, N_PER_TILE // LANES, zbody, 0, unroll=8)
    pltpu.sync_copy(zero_v, acc_sh.at[pl.ds(sid * N_PER_TILE, N_PER_TILE)])
    plsc.subcore_barrier()

    pltpu.sync_copy(m_hbm.at[wid], val_v)
    pltpu.sync_copy(edst_hbm.at[wid], idx_v)

    def sbody(j, carry):
        pltpu.sync_copy(val_v.at[j], acc_sh.at[idx_v.at[j]], add=True)
        return carry

    lax.fori_loop(0, ROWS_PER_W, sbody, 0)
    plsc.subcore_barrier()

    @pl.when(sid == 0)
    def _():
        pltpu.sync_copy(acc_sh, out_hbm.at[cid])


def _mlp_body(p_ref, hs_ref, hd_ref, w_ref, o_ref):
    hs = hs_ref[...]
    hd = hd_ref[...]
    w = w_ref[...]

    def step(k, acc):
        t = hs * p_ref[0, k] + hd * p_ref[1, k] + w * p_ref[2, k] + p_ref[3, k]
        u = t + 0.044715 * (t * t * t)
        g = 0.5 * t * (1.0 + jnp.tanh(0.7978845608028654 * u))
        return acc + g * p_ref[4, k]

    acc = lax.fori_loop(0, HIDDEN, step, jnp.zeros_like(hs), unroll=8)
    o_ref[...] = (acc + p_ref[5, 0]) * w


_MLP_BLK = 392


def _tc_mlp(p, hs2, hd2, w2):
    return pl.pallas_call(
        _mlp_body,
        grid=(ROWS // _MLP_BLK,),
        in_specs=[
            pl.BlockSpec(memory_space=pltpu.SMEM),
            pl.BlockSpec((_MLP_BLK, 128), lambda i: (i, 0)),
            pl.BlockSpec((_MLP_BLK, 128), lambda i: (i, 0)),
            pl.BlockSpec((_MLP_BLK, 128), lambda i: (i, 0)),
        ],
        out_specs=pl.BlockSpec((_MLP_BLK, 128), lambda i: (i, 0)),
        out_shape=jax.ShapeDtypeStruct((ROWS, 128), jnp.float32),
    )(p, hs2, hd2, w2)


def _upd_body(final, sb_ref, h_ref, a0_ref, a1_ref, d_ref, o_ref):
    x = h_ref[...] + (a0_ref[...] + a1_ref[...]) / d_ref[...]
    mu = x
    dl = x - mu
    var = dl * dl
    y = dl / jnp.sqrt(var + 1e-6) * sb_ref[0, 0] + sb_ref[0, 1]
    if final:
        y = jnp.maximum(y, 0.0) + jnp.log1p(jnp.exp(-jnp.abs(y)))
    o_ref[...] = y


_NROWS = N_NODES // 128


def _tc_update(lslb, h2, a0, a1, d2, final):
    return pl.pallas_call(
        functools.partial(_upd_body, final),
        in_specs=[
            pl.BlockSpec(memory_space=pltpu.SMEM),
            pl.BlockSpec((_NROWS, 128), lambda: (0, 0)),
            pl.BlockSpec((_NROWS, 128), lambda: (0, 0)),
            pl.BlockSpec((_NROWS, 128), lambda: (0, 0)),
            pl.BlockSpec((_NROWS, 128), lambda: (0, 0)),
        ],
        out_specs=pl.BlockSpec((_NROWS, 128), lambda: (0, 0)),
        out_shape=jax.ShapeDtypeStruct((_NROWS, 128), jnp.float32),
    )(lslb, h2, a0, a1, d2)


def kernel(eps_2d, esrc, edst, ew, ndeg,
           W1_0, b1_0, W2_0, b2_0, ln_s_0, ln_b_0,
           W1_1, b1_1, W2_1, b2_1, ln_s_1, ln_b_1):
    params = [
        (W1_0, b1_0, W2_0, b2_0, ln_s_0, ln_b_0),
        (W1_1, b1_1, W2_1, b2_1, ln_s_1, ln_b_1),
    ]
    h = eps_2d.reshape(-1)
    ew2 = ew.reshape(ROWS, 128)
    edst3 = edst.reshape(NW, ROWS_PER_W, 128)
    d2 = ndeg.reshape(_NROWS, 128)

    for li, (W1, b1, W2, b2, ls, lb) in enumerate(params):
        p = jnp.stack([W1[0], W1[1], W1[2], b1, W2[:, 0],
                       jnp.broadcast_to(b2, (HIDDEN,))])
        lslb = jnp.stack([ls[0], lb[0]]).reshape(1, 2)
        hs, hd = _sc_gather(h, esrc, edst)
        m2 = _tc_mlp(p, hs.reshape(ROWS, 128), hd.reshape(ROWS, 128), ew2)
        agg = _sc_scatter(m2.reshape(NW, ROWS_PER_W, 128), edst3)
        h2 = _tc_update(lslb, h.reshape(_NROWS, 128),
                        agg[0].reshape(_NROWS, 128),
                        agg[1].reshape(_NROWS, 128),
                        d2, final=(li == 1))
        h = h2.reshape(-1)

    return h.reshape(N_GRID, N_GRID)


# speedup vs baseline: 62.5689x; 1.0803x over previous
import functools

import jax
import jax.numpy as jnp
from jax import lax
from jax.experimental import pallas as pl
from jax.experimental.pallas import tpu as pltpu
from jax.experimental.pallas import tpu_sc as plsc

N_GRID = 224
N_NODES = N_GRID * N_GRID
E_TOTAL = N_NODES * 16
HIDDEN = 96
LANES = 16
NC, NS = 2, 16
NW = NC * NS
E_PER_W = E_TOTAL // NW
GV = E_PER_W // LANES
ROWS = E_TOTAL // 128
ROWS_PER_W = ROWS // NW
N_PER_TILE = N_NODES // NS

_mesh = plsc.VectorSubcoreMesh(core_axis_name="c", subcore_axis_name="s")
_sc_params = pltpu.CompilerParams(needs_layout_passes=False)


@functools.partial(
    pl.kernel,
    out_type=[jax.ShapeDtypeStruct((E_TOTAL,), jnp.float32),
              jax.ShapeDtypeStruct((E_TOTAL,), jnp.float32)],
    mesh=_mesh,
    scratch_types=[pltpu.VMEM((N_NODES,), jnp.float32),
                   pltpu.VMEM((E_PER_W,), jnp.int32),
                   pltpu.VMEM((E_PER_W,), jnp.float32)],
    compiler_params=_sc_params,
)
def _sc_gather(h_hbm, esrc_hbm, edst_hbm, hs_out, hd_out, h_v, idx_v, out_v):
    wid = lax.axis_index("s") * NC + lax.axis_index("c")
    base = wid * E_PER_W
    pltpu.sync_copy(h_hbm, h_v)

    def one_pass(idx_hbm, o_hbm):
        pltpu.sync_copy(idx_hbm.at[pl.ds(base, E_PER_W)], idx_v)

        @plsc.parallel_loop(0, GV, unroll=8)
        def body(i):
            iv = idx_v[pl.ds(i * LANES, LANES)]
            out_v[pl.ds(i * LANES, LANES)] = plsc.load_gather(h_v, [iv])
        pltpu.sync_copy(out_v, o_hbm.at[pl.ds(base, E_PER_W)])

    one_pass(esrc_hbm, hs_out)
    one_pass(edst_hbm, hd_out)


@functools.partial(
    pl.kernel,
    out_type=jax.ShapeDtypeStruct((NC, N_NODES), jnp.float32),
    mesh=_mesh,
    scratch_types=[pltpu.VMEM((ROWS_PER_W, 128), jnp.int32),
                   pltpu.VMEM((ROWS_PER_W, 128), jnp.float32),
                   pltpu.VMEM((N_PER_TILE,), jnp.float32),
                   pltpu.VMEM_SHARED((N_NODES,), jnp.float32)],
    compiler_params=_sc_params,
)
def _sc_scatter(m_hbm, edst_hbm, out_hbm, idx_v, val_v, zero_v, acc_sh):
    cid = lax.axis_index("c")
    sid = lax.axis_index("s")
    wid = sid * NC + cid

    @plsc.parallel_loop(0, N_PER_TILE // LANES, unroll=8)
    def zbody(i):
        zero_v[pl.ds(i * LANES, LANES)] = jnp.zeros((LANES,), jnp.float32)
    pltpu.sync_copy(zero_v, acc_sh.at[pl.ds(sid * N_PER_TILE, N_PER_TILE)])
    plsc.subcore_barrier()

    pltpu.sync_copy(m_hbm.at[wid], val_v)
    pltpu.sync_copy(edst_hbm.at[wid], idx_v)

    @plsc.parallel_loop(0, ROWS_PER_W, unroll=4)
    def sbody(j):
        pltpu.sync_copy(val_v.at[j], acc_sh.at[idx_v.at[j]], add=True)
    plsc.subcore_barrier()

    @pl.when(sid == 0)
    def _():
        pltpu.sync_copy(acc_sh, out_hbm.at[cid])


def _mlp_body(p_ref, hs_ref, hd_ref, w_ref, o_ref):
    hs = hs_ref[...]
    hd = hd_ref[...]
    w = w_ref[...]

    def step(k, acc):
        t = hs * p_ref[0, k] + hd * p_ref[1, k] + w * p_ref[2, k] + p_ref[3, k]
        u = t + 0.044715 * (t * t * t)
        g = 0.5 * t * (1.0 + jnp.tanh(0.7978845608028654 * u))
        return acc + g * p_ref[4, k]

    acc = lax.fori_loop(0, HIDDEN, step, jnp.zeros_like(hs), unroll=8)
    o_ref[...] = (acc + p_ref[5, 0]) * w


_MLP_BLK = 392


def _tc_mlp(p, hs2, hd2, w2):
    return pl.pallas_call(
        _mlp_body,
        grid=(ROWS // _MLP_BLK,),
        in_specs=[
            pl.BlockSpec(memory_space=pltpu.SMEM),
            pl.BlockSpec((_MLP_BLK, 128), lambda i: (i, 0)),
            pl.BlockSpec((_MLP_BLK, 128), lambda i: (i, 0)),
            pl.BlockSpec((_MLP_BLK, 128), lambda i: (i, 0)),
        ],
        out_specs=pl.BlockSpec((_MLP_BLK, 128), lambda i: (i, 0)),
        out_shape=jax.ShapeDtypeStruct((ROWS, 128), jnp.float32),
    )(p, hs2, hd2, w2)


def _upd_body(final, sb_ref, h_ref, a0_ref, a1_ref, d_ref, o_ref):
    x = h_ref[...] + (a0_ref[...] + a1_ref[...]) / d_ref[...]
    mu = x
    dl = x - mu
    var = dl * dl
    y = dl / jnp.sqrt(var + 1e-6) * sb_ref[0, 0] + sb_ref[0, 1]
    if final:
        y = jnp.maximum(y, 0.0) + jnp.log1p(jnp.exp(-jnp.abs(y)))
    o_ref[...] = y


_NROWS = N_NODES // 128


def _tc_update(lslb, h2, a0, a1, d2, final):
    return pl.pallas_call(
        functools.partial(_upd_body, final),
        in_specs=[
            pl.BlockSpec(memory_space=pltpu.SMEM),
            pl.BlockSpec((_NROWS, 128), lambda: (0, 0)),
            pl.BlockSpec((_NROWS, 128), lambda: (0, 0)),
            pl.BlockSpec((_NROWS, 128), lambda: (0, 0)),
            pl.BlockSpec((_NROWS, 128), lambda: (0, 0)),
        ],
        out_specs=pl.BlockSpec((_NROWS, 128), lambda: (0, 0)),
        out_shape=jax.ShapeDtypeStruct((_NROWS, 128), jnp.float32),
    )(lslb, h2, a0, a1, d2)


def kernel(eps_2d, esrc, edst, ew, ndeg,
           W1_0, b1_0, W2_0, b2_0, ln_s_0, ln_b_0,
           W1_1, b1_1, W2_1, b2_1, ln_s_1, ln_b_1):
    params = [
        (W1_0, b1_0, W2_0, b2_0, ln_s_0, ln_b_0),
        (W1_1, b1_1, W2_1, b2_1, ln_s_1, ln_b_1),
    ]
    h = eps_2d.reshape(-1)
    ew2 = ew.reshape(ROWS, 128)
    edst3 = edst.reshape(NW, ROWS_PER_W, 128)
    d2 = ndeg.reshape(_NROWS, 128)

    for li, (W1, b1, W2, b2, ls, lb) in enumerate(params):
        p = jnp.stack([W1[0], W1[1], W1[2], b1, W2[:, 0],
                       jnp.broadcast_to(b2, (HIDDEN,))])
        lslb = jnp.stack([ls[0], lb[0]]).reshape(1, 2)
        hs, hd = _sc_gather(h, esrc, edst)
        m2 = _tc_mlp(p, hs.reshape(ROWS, 128), hd.reshape(ROWS, 128), ew2)
        agg = _sc_scatter(m2.reshape(NW, ROWS_PER_W, 128), edst3)
        h2 = _tc_update(lslb, h.reshape(_NROWS, 128),
                        agg[0].reshape(_NROWS, 128),
                        agg[1].reshape(_NROWS, 128),
                        d2, final=(li == 1))
        h = h2.reshape(-1)

    return h.reshape(N_GRID, N_GRID)


# speedup vs baseline: 65.9554x; 1.0541x over previous
import functools

import jax
import jax.numpy as jnp
from jax import lax
from jax.experimental import pallas as pl
from jax.experimental.pallas import tpu as pltpu
from jax.experimental.pallas import tpu_sc as plsc

N_GRID = 224
N_NODES = N_GRID * N_GRID
E_TOTAL = N_NODES * 16
HIDDEN = 96
LANES = 16
NC, NS = 2, 16
NW = NC * NS
NCHUNK = 2
E_C = E_TOTAL // NCHUNK
E_PER_W = E_C // NW
GV = E_PER_W // LANES
ROWS_C = E_C // 128
ROWS_PER_W = ROWS_C // NW
N_PER_TILE = N_NODES // NS

_mesh = plsc.VectorSubcoreMesh(core_axis_name="c", subcore_axis_name="s")
_sc_params = pltpu.CompilerParams(needs_layout_passes=False)


@functools.partial(
    pl.kernel,
    out_type=[jax.ShapeDtypeStruct((E_C,), jnp.float32),
              jax.ShapeDtypeStruct((E_C,), jnp.float32)],
    mesh=_mesh,
    scratch_types=[pltpu.VMEM((N_NODES,), jnp.float32),
                   pltpu.VMEM((E_PER_W,), jnp.int32),
                   pltpu.VMEM((E_PER_W,), jnp.float32)],
    compiler_params=_sc_params,
)
def _sc_gather(h_hbm, esrc_hbm, edst_hbm, hs_out, hd_out, h_v, idx_v, out_v):
    wid = lax.axis_index("s") * NC + lax.axis_index("c")
    base = wid * E_PER_W
    pltpu.sync_copy(h_hbm, h_v)

    def one_pass(idx_hbm, o_hbm):
        pltpu.sync_copy(idx_hbm.at[pl.ds(base, E_PER_W)], idx_v)

        @plsc.parallel_loop(0, GV, unroll=8)
        def body(i):
            iv = idx_v[pl.ds(i * LANES, LANES)]
            out_v[pl.ds(i * LANES, LANES)] = plsc.load_gather(h_v, [iv])

        pltpu.sync_copy(out_v, o_hbm.at[pl.ds(base, E_PER_W)])

    one_pass(esrc_hbm, hs_out)
    one_pass(edst_hbm, hd_out)


@functools.partial(
    pl.kernel,
    out_type=jax.ShapeDtypeStruct((NC, N_NODES), jnp.float32),
    mesh=_mesh,
    scratch_types=[pltpu.VMEM((ROWS_PER_W, 128), jnp.int32),
                   pltpu.VMEM((ROWS_PER_W, 128), jnp.float32),
                   pltpu.VMEM((N_PER_TILE,), jnp.float32),
                   pltpu.VMEM_SHARED((N_NODES,), jnp.float32)],
    compiler_params=_sc_params,
)
def _sc_scatter(m_hbm, edst_hbm, out_hbm, idx_v, val_v, zero_v, acc_sh):
    cid = lax.axis_index("c")
    sid = lax.axis_index("s")
    wid = sid * NC + cid

    @plsc.parallel_loop(0, N_PER_TILE // LANES, unroll=8)
    def zbody(i):
        zero_v[pl.ds(i * LANES, LANES)] = jnp.zeros((LANES,), jnp.float32)

    pltpu.sync_copy(zero_v, acc_sh.at[pl.ds(sid * N_PER_TILE, N_PER_TILE)])
    plsc.subcore_barrier()

    pltpu.sync_copy(m_hbm.at[wid], val_v)
    pltpu.sync_copy(edst_hbm.at[wid], idx_v)

    @plsc.parallel_loop(0, ROWS_PER_W, unroll=4)
    def sbody(j):
        pltpu.sync_copy(val_v.at[j], acc_sh.at[idx_v.at[j]], add=True)

    plsc.subcore_barrier()

    @pl.when(sid == 0)
    def _():
        pltpu.sync_copy(acc_sh, out_hbm.at[cid])


def _mlp_body(p_ref, hs_ref, hd_ref, w_ref, o_ref):
    def outer(s, c):
        sl = pl.ds(s * 8, 8)
        hs = hs_ref[sl, :]
        hd = hd_ref[sl, :]
        w = w_ref[sl, :]
        acc = jnp.zeros((8, 128), jnp.float32)
        for k in range(HIDDEN):
            t = (hs * p_ref[0, k] + hd * p_ref[1, k]
                 + w * p_ref[2, k] + p_ref[3, k])
            u = t + 0.044715 * (t * t * t)
            g = t * (1.0 + jnp.tanh(0.7978845608028654 * u))
            acc = acc + g * p_ref[4, k]
        o_ref[sl, :] = (acc + p_ref[5, 0]) * w
        return c

    lax.fori_loop(0, _MLP_BLK // 8, outer, 0)


_MLP_BLK = 392


def _tc_mlp(p, hs2, hd2, w2):
    return pl.pallas_call(
        _mlp_body,
        grid=(ROWS_C // _MLP_BLK,),
        in_specs=[
            pl.BlockSpec(memory_space=pltpu.SMEM),
            pl.BlockSpec((_MLP_BLK, 128), lambda i: (i, 0)),
            pl.BlockSpec((_MLP_BLK, 128), lambda i: (i, 0)),
            pl.BlockSpec((_MLP_BLK, 128), lambda i: (i, 0)),
        ],
        out_specs=pl.BlockSpec((_MLP_BLK, 128), lambda i: (i, 0)),
        out_shape=jax.ShapeDtypeStruct((ROWS_C, 128), jnp.float32),
    )(p, hs2, hd2, w2)


def _upd_body(final, sb_ref, h_ref, a_ref, b_ref, d_ref, o_ref):
    av = a_ref[...]
    bv = b_ref[...]
    agg = (av[:_NROWS] + av[_NROWS:]) + (bv[:_NROWS] + bv[_NROWS:])
    x = h_ref[...] + agg / d_ref[...]
    mu = x
    dl = x - mu
    var = dl * dl
    y = dl / jnp.sqrt(var + 1e-6) * sb_ref[0, 0] + sb_ref[0, 1]
    if final:
        y = jnp.maximum(y, 0.0) + jnp.log1p(jnp.exp(-jnp.abs(y)))
    o_ref[...] = y


_NROWS = N_NODES // 128


def _tc_update(lslb, h2, agg_a, agg_b, d2, final):
    return pl.pallas_call(
        functools.partial(_upd_body, final),
        in_specs=[
            pl.BlockSpec(memory_space=pltpu.SMEM),
            pl.BlockSpec((_NROWS, 128), lambda: (0, 0)),
            pl.BlockSpec((2 * _NROWS, 128), lambda: (0, 0)),
            pl.BlockSpec((2 * _NROWS, 128), lambda: (0, 0)),
            pl.BlockSpec((_NROWS, 128), lambda: (0, 0)),
        ],
        out_specs=pl.BlockSpec((_NROWS, 128), lambda: (0, 0)),
        out_shape=jax.ShapeDtypeStruct((_NROWS, 128), jnp.float32),
    )(lslb, h2, agg_a, agg_b, d2)


def kernel(eps_2d, esrc, edst, ew, ndeg,
           W1_0, b1_0, W2_0, b2_0, ln_s_0, ln_b_0,
           W1_1, b1_1, W2_1, b2_1, ln_s_1, ln_b_1):
    params = [
        (W1_0, b1_0, W2_0, b2_0, ln_s_0, ln_b_0),
        (W1_1, b1_1, W2_1, b2_1, ln_s_1, ln_b_1),
    ]
    h = eps_2d.reshape(-1)
    d2 = ndeg.reshape(_NROWS, 128)
    esrc_c = [lax.slice(esrc, (i * E_C,), ((i + 1) * E_C,)) for i in range(NCHUNK)]
    edst_c = [lax.slice(edst, (i * E_C,), ((i + 1) * E_C,)) for i in range(NCHUNK)]
    ew_c = [lax.slice(ew, (i * E_C,), ((i + 1) * E_C,)).reshape(ROWS_C, 128)
            for i in range(NCHUNK)]
    edst3_c = [e.reshape(NW, ROWS_PER_W, 128) for e in edst_c]

    for li, (W1, b1, W2, b2, ls, lb) in enumerate(params):
        p = jnp.stack([W1[0], W1[1], W1[2], b1, 0.5 * W2[:, 0],
                       jnp.broadcast_to(b2, (HIDDEN,))])
        lslb = jnp.stack([ls[0], lb[0]]).reshape(1, 2)
        gath = [_sc_gather(h, esrc_c[i], edst_c[i]) for i in range(NCHUNK)]
        msgs = [_tc_mlp(p, gath[i][0].reshape(ROWS_C, 128),
                        gath[i][1].reshape(ROWS_C, 128), ew_c[i])
                for i in range(NCHUNK)]
        aggs = [_sc_scatter(msgs[i].reshape(NW, ROWS_PER_W, 128), edst3_c[i])
                for i in range(NCHUNK)]
        h2 = _tc_update(lslb, h.reshape(_NROWS, 128),
                        aggs[0].reshape(2 * _NROWS, 128),
                        aggs[1].reshape(2 * _NROWS, 128),
                        d2, final=(li == 1))
        h = h2.reshape(-1)

    return h.reshape(N_GRID, N_GRID)


# speedup vs baseline: 67.9007x; 1.0295x over previous
import functools

import jax
import jax.numpy as jnp
from jax import lax
from jax.experimental import pallas as pl
from jax.experimental.pallas import tpu as pltpu
from jax.experimental.pallas import tpu_sc as plsc

N_GRID = 224
N_NODES = N_GRID * N_GRID
E_TOTAL = N_NODES * 16
HIDDEN = 96
LANES = 16
NC, NS = 2, 16
NW = NC * NS
NCHUNK = 2
E_C = E_TOTAL // NCHUNK
E_PER_W = E_C // NW
GV = E_PER_W // LANES
ROWS_C = E_C // 128
ROWS_PER_W = ROWS_C // NW
N_PER_TILE = N_NODES // NS

_mesh = plsc.VectorSubcoreMesh(core_axis_name="c", subcore_axis_name="s")
_sc_params = pltpu.CompilerParams(needs_layout_passes=False)


@functools.partial(
    pl.kernel,
    out_type=[jax.ShapeDtypeStruct((E_C,), jnp.float32),
              jax.ShapeDtypeStruct((E_C,), jnp.float32)],
    mesh=_mesh,
    scratch_types=[pltpu.VMEM((N_NODES,), jnp.float32),
                   pltpu.VMEM((E_PER_W,), jnp.int32),
                   pltpu.VMEM((E_PER_W,), jnp.float32)],
    compiler_params=_sc_params,
)
def _sc_gather(h_hbm, esrc_hbm, edst_hbm, hs_out, hd_out, h_v, idx_v, out_v):
    wid = lax.axis_index("s") * NC + lax.axis_index("c")
    base = wid * E_PER_W
    pltpu.sync_copy(h_hbm, h_v)

    def one_pass(idx_hbm, o_hbm):
        pltpu.sync_copy(idx_hbm.at[pl.ds(base, E_PER_W)], idx_v)

        @plsc.parallel_loop(0, GV, unroll=8)
        def body(i):
            iv = idx_v[pl.ds(i * LANES, LANES)]
            out_v[pl.ds(i * LANES, LANES)] = plsc.load_gather(h_v, [iv])

        pltpu.sync_copy(out_v, o_hbm.at[pl.ds(base, E_PER_W)])

    one_pass(esrc_hbm, hs_out)
    one_pass(edst_hbm, hd_out)


@functools.partial(
    pl.kernel,
    out_type=jax.ShapeDtypeStruct((NC, N_NODES), jnp.float32),
    mesh=_mesh,
    scratch_types=[pltpu.VMEM((ROWS_PER_W, 128), jnp.int32),
                   pltpu.VMEM((ROWS_PER_W, 128), jnp.float32),
                   pltpu.VMEM((N_PER_TILE,), jnp.float32),
                   pltpu.VMEM_SHARED((N_NODES,), jnp.float32)],
    compiler_params=_sc_params,
)
def _sc_scatter(m_hbm, edst_hbm, out_hbm, idx_v, val_v, zero_v, acc_sh):
    cid = lax.axis_index("c")
    sid = lax.axis_index("s")
    wid = sid * NC + cid

    @plsc.parallel_loop(0, N_PER_TILE // LANES, unroll=8)
    def zbody(i):
        zero_v[pl.ds(i * LANES, LANES)] = jnp.zeros((LANES,), jnp.float32)

    pltpu.sync_copy(zero_v, acc_sh.at[pl.ds(sid * N_PER_TILE, N_PER_TILE)])
    plsc.subcore_barrier()

    pltpu.sync_copy(m_hbm.at[wid], val_v)
    pltpu.sync_copy(edst_hbm.at[wid], idx_v)

    @plsc.parallel_loop(0, ROWS_PER_W, unroll=4)
    def sbody(j):
        pltpu.sync_copy(val_v.at[j], acc_sh.at[idx_v.at[j]], add=True)

    plsc.subcore_barrier()

    @pl.when(sid == 0)
    def _():
        pltpu.sync_copy(acc_sh, out_hbm.at[cid])


def _mlp_body(p_ref, hs_ref, hd_ref, w_ref, o_ref):
    def outer(s, c):
        sl = pl.ds(s * 16, 16)
        hs = hs_ref[sl, :]
        hd = hd_ref[sl, :]
        w = w_ref[sl, :]
        acc = jnp.zeros((16, 128), jnp.float32)
        for k in range(HIDDEN):
            t = (hs * p_ref[0, k] + hd * p_ref[1, k]
                 + w * p_ref[2, k] + p_ref[3, k])
            u = t + 0.044715 * (t * t * t)
            g = t * (1.0 + jnp.tanh(0.7978845608028654 * u))
            acc = acc + g * p_ref[4, k]
        o_ref[sl, :] = (acc + p_ref[5, 0]) * w
        return c

    lax.fori_loop(0, _MLP_BLK // 16, outer, 0)


_MLP_BLK = 448


def _tc_mlp(p, hs2, hd2, w2):
    return pl.pallas_call(
        _mlp_body,
        grid=(ROWS_C // _MLP_BLK,),
        in_specs=[
            pl.BlockSpec(memory_space=pltpu.SMEM),
            pl.BlockSpec((_MLP_BLK, 128), lambda i: (i, 0)),
            pl.BlockSpec((_MLP_BLK, 128), lambda i: (i, 0)),
            pl.BlockSpec((_MLP_BLK, 128), lambda i: (i, 0)),
        ],
        out_specs=pl.BlockSpec((_MLP_BLK, 128), lambda i: (i, 0)),
        out_shape=jax.ShapeDtypeStruct((ROWS_C, 128), jnp.float32),
    )(p, hs2, hd2, w2)


def _upd_body(final, sb_ref, h_ref, a_ref, b_ref, d_ref, o_ref):
    av = a_ref[...]
    bv = b_ref[...]
    agg = (av[:_NROWS] + av[_NROWS:]) + (bv[:_NROWS] + bv[_NROWS:])
    x = h_ref[...] + agg / d_ref[...]
    mu = x
    dl = x - mu
    var = dl * dl
    y = dl / jnp.sqrt(var + 1e-6) * sb_ref[0, 0] + sb_ref[0, 1]
    if final:
        y = jnp.maximum(y, 0.0) + jnp.log1p(jnp.exp(-jnp.abs(y)))
    o_ref[...] = y


_NROWS = N_NODES // 128


def _tc_update(lslb, h2, agg_a, agg_b, d2, final):
    return pl.pallas_call(
        functools.partial(_upd_body, final),
        in_specs=[
            pl.BlockSpec(memory_space=pltpu.SMEM),
            pl.BlockSpec((_NROWS, 128), lambda: (0, 0)),
            pl.BlockSpec((2 * _NROWS, 128), lambda: (0, 0)),
            pl.BlockSpec((2 * _NROWS, 128), lambda: (0, 0)),
            pl.BlockSpec((_NROWS, 128), lambda: (0, 0)),
        ],
        out_specs=pl.BlockSpec((_NROWS, 128), lambda: (0, 0)),
        out_shape=jax.ShapeDtypeStruct((_NROWS, 128), jnp.float32),
    )(lslb, h2, agg_a, agg_b, d2)


def kernel(eps_2d, esrc, edst, ew, ndeg,
           W1_0, b1_0, W2_0, b2_0, ln_s_0, ln_b_0,
           W1_1, b1_1, W2_1, b2_1, ln_s_1, ln_b_1):
    params = [
        (W1_0, b1_0, W2_0, b2_0, ln_s_0, ln_b_0),
        (W1_1, b1_1, W2_1, b2_1, ln_s_1, ln_b_1),
    ]
    h = eps_2d.reshape(-1)
    d2 = ndeg.reshape(_NROWS, 128)
    esrc_c = [lax.slice(esrc, (i * E_C,), ((i + 1) * E_C,)) for i in range(NCHUNK)]
    edst_c = [lax.slice(edst, (i * E_C,), ((i + 1) * E_C,)) for i in range(NCHUNK)]
    ew_c = [lax.slice(ew, (i * E_C,), ((i + 1) * E_C,)).reshape(ROWS_C, 128)
            for i in range(NCHUNK)]
    edst3_c = [e.reshape(NW, ROWS_PER_W, 128) for e in edst_c]

    for li, (W1, b1, W2, b2, ls, lb) in enumerate(params):
        p = jnp.stack([W1[0], W1[1], W1[2], b1, 0.5 * W2[:, 0],
                       jnp.broadcast_to(b2, (HIDDEN,))])
        lslb = jnp.stack([ls[0], lb[0]]).reshape(1, 2)
        gath = [_sc_gather(h, esrc_c[i], edst_c[i]) for i in range(NCHUNK)]
        msgs = [_tc_mlp(p, gath[i][0].reshape(ROWS_C, 128),
                        gath[i][1].reshape(ROWS_C, 128), ew_c[i])
                for i in range(NCHUNK)]
        aggs = [_sc_scatter(msgs[i].reshape(NW, ROWS_PER_W, 128), edst3_c[i])
                for i in range(NCHUNK)]
        h2 = _tc_update(lslb, h.reshape(_NROWS, 128),
                        aggs[0].reshape(2 * _NROWS, 128),
                        aggs[1].reshape(2 * _NROWS, 128),
                        d2, final=(li == 1))
        h = h2.reshape(-1)

    return h.reshape(N_GRID, N_GRID)
